# baseline (speedup 1.0000x reference)
import jax
import jax.numpy as jnp
from jax import lax
from jax.experimental import pallas as pl
from jax.experimental.pallas import tpu as pltpu, tpu_sc as plsc

_B = 16384
_D = 64

_NC, _NS = 2, 16
_NW = _NC * _NS
_BPW = _B // _NW


def _gather_body(users_hbm, movies_hbm, ut_hbm, mt_hbm, outu_hbm, outm_hbm,
                 idx_u, idx_m, rows_u, rows_m, sem_u, sem_m):
    wid = lax.axis_index("s") * _NC + lax.axis_index("c")
    base = wid * _BPW
    pltpu.sync_copy(users_hbm.at[pl.ds(base, _BPW)], idx_u)
    pltpu.sync_copy(movies_hbm.at[pl.ds(base, _BPW)], idx_m)
    cu = pltpu.async_copy(ut_hbm.at[idx_u], rows_u, sem_u)
    cm = pltpu.async_copy(mt_hbm.at[idx_m], rows_m, sem_m)
    cu.wait()
    cm.wait()
    pltpu.sync_copy(rows_u, outu_hbm.at[pl.ds(base, _BPW)])
    pltpu.sync_copy(rows_m, outm_hbm.at[pl.ds(base, _BPW)])


def _sc_gather(users, movies, user_table, movie_table):
    mesh = plsc.VectorSubcoreMesh(core_axis_name="c", subcore_axis_name="s")
    return pl.kernel(
        _gather_body,
        mesh=mesh,
        compiler_params=pltpu.CompilerParams(use_tc_tiling_on_sc=False),
        out_type=[jax.ShapeDtypeStruct((_B, _D), jnp.float32),
                  jax.ShapeDtypeStruct((_B, _D), jnp.float32)],
        scratch_types=[
            pltpu.VMEM((_BPW,), jnp.int32),
            pltpu.VMEM((_BPW,), jnp.int32),
            pltpu.VMEM((_BPW, _D), jnp.float32),
            pltpu.VMEM((_BPW, _D), jnp.float32),
            pltpu.SemaphoreType.DMA,
            pltpu.SemaphoreType.DMA,
        ],
    )(users, movies, user_table, movie_table)


_BS = 2048


def _mlp_body(xu_ref, xm_ref, w1_ref, b1_ref, w2_ref, b2_ref, w3_ref, b3_ref,
              w4_ref, b4_ref, out_ref):
    h = jnp.maximum(
        jnp.dot(xu_ref[...], w1_ref[0:_D, :],
                preferred_element_type=jnp.float32)
        + jnp.dot(xm_ref[...], w1_ref[_D:2 * _D, :],
                  preferred_element_type=jnp.float32)
        + b1_ref[...], 0.0)
    h = jnp.maximum(
        jnp.dot(h, w2_ref[...], preferred_element_type=jnp.float32)
        + b2_ref[...], 0.0)
    h = jnp.maximum(
        jnp.dot(h, w3_ref[...], preferred_element_type=jnp.float32)
        + b3_ref[...], 0.0)
    logit = jnp.sum(h * w4_ref[...], axis=1) + b4_ref[0, 0]
    out_ref[...] = 1.0 / (1.0 + jnp.exp(-logit))


def _tc_mlp(xu, xm, W1, b1, W2, b2, W3, b3, W4, b4):
    grid = (_B // _BS,)
    full = lambda shape: pl.BlockSpec(shape, lambda i: (0,) * len(shape))
    return pl.pallas_call(
        _mlp_body,
        grid=grid,
        in_specs=[
            pl.BlockSpec((_BS, _D), lambda i: (i, 0)),
            pl.BlockSpec((_BS, _D), lambda i: (i, 0)),
            full((2 * _D, 256)), full((1, 256)),
            full((256, 128)), full((1, 128)),
            full((128, _D)), full((1, _D)),
            full((1, _D)), full((1, 1)),
        ],
        out_specs=pl.BlockSpec((_BS,), lambda i: (i,)),
        out_shape=jax.ShapeDtypeStruct((_B,), jnp.float32),
    )(xu, xm, W1, b1.reshape(1, 256), W2, b2.reshape(1, 128),
      W3, b3.reshape(1, _D), W4.reshape(1, _D), b4.reshape(1, 1))


def kernel(users, movies, user_table, movie_table,
           W1, b1, W2, b2, W3, b3, W4, b4):
    xu, xm = _sc_gather(users.astype(jnp.int32), movies.astype(jnp.int32),
                        user_table, movie_table)
    return _tc_mlp(xu, xm, W1, b1, W2, b2, W3, b3, W4, b4)


# speedup vs baseline: 1.0065x; 1.0065x over previous
import jax
import jax.numpy as jnp
from jax import lax
from jax.experimental import pallas as pl
from jax.experimental.pallas import tpu as pltpu, tpu_sc as plsc

_B = 16384
_D = 64

_NC, _NS = 2, 16
_NW = _NC * _NS
_BPW = _B // _NW
_LANES = 16


def _gather_body(users_hbm, movies_hbm, ut2_hbm, mt2_hbm, outu_hbm, outm_hbm,
                 idx, half, rows, sem):
    wid = lax.axis_index("s") * _NC + lax.axis_index("c")
    base = wid * _BPW

    def one_table(src_idx_hbm, table_hbm, out_hbm):
        pltpu.sync_copy(src_idx_hbm.at[pl.ds(base, _BPW)], idx)
        def halve(v, _):
            half[pl.ds(v * _LANES, _LANES)] = (
                idx[pl.ds(v * _LANES, _LANES)] >> 1)
            return 0
        lax.fori_loop(0, _BPW // _LANES, halve, 0, unroll=8)
        pltpu.async_copy(table_hbm.at[half], rows, sem).wait()
        pltpu.sync_copy(rows, out_hbm.at[pl.ds(base, _BPW)])

    one_table(users_hbm, ut2_hbm, outu_hbm)
    one_table(movies_hbm, mt2_hbm, outm_hbm)


def _sc_gather(users, movies, ut2, mt2):
    mesh = plsc.VectorSubcoreMesh(core_axis_name="c", subcore_axis_name="s")
    return pl.kernel(
        _gather_body,
        mesh=mesh,
        out_type=[jax.ShapeDtypeStruct((_B, 2 * _D), jnp.float32),
                  jax.ShapeDtypeStruct((_B, 2 * _D), jnp.float32)],
        scratch_types=[
            pltpu.VMEM((_BPW,), jnp.int32),
            pltpu.VMEM((_BPW,), jnp.int32),
            pltpu.VMEM((_BPW, 2 * _D), jnp.float32),
            pltpu.SemaphoreType.DMA,
        ],
    )(users, movies, ut2, mt2)


_BS = 2048


def _mlp_body(gu_ref, gm_ref, pu_ref, pm_ref, w1_ref, b1_ref, w2_ref, b2_ref,
              w3_ref, b3_ref, w4_ref, b4_ref, out_ref):
    pu = (pu_ref[...] & 1) == 1
    pm = (pm_ref[...] & 1) == 1
    xu = jnp.where(pu, gu_ref[:, _D:], gu_ref[:, :_D])
    xm = jnp.where(pm, gm_ref[:, _D:], gm_ref[:, :_D])
    h = jnp.maximum(
        jnp.dot(xu, w1_ref[0:_D, :], preferred_element_type=jnp.float32)
        + jnp.dot(xm, w1_ref[_D:2 * _D, :],
                  preferred_element_type=jnp.float32)
        + b1_ref[...], 0.0)
    h = jnp.maximum(
        jnp.dot(h, w2_ref[...], preferred_element_type=jnp.float32)
        + b2_ref[...], 0.0)
    h = jnp.maximum(
        jnp.dot(h, w3_ref[...], preferred_element_type=jnp.float32)
        + b3_ref[...], 0.0)
    logit = jnp.sum(h * w4_ref[...], axis=1) + b4_ref[0, 0]
    out_ref[...] = 1.0 / (1.0 + jnp.exp(-logit))


def _tc_mlp(gu, gm, users2d, movies2d, W1, b1, W2, b2, W3, b3, W4, b4):
    grid = (_B // _BS,)
    full = lambda shape: pl.BlockSpec(shape, lambda i: (0,) * len(shape))
    return pl.pallas_call(
        _mlp_body,
        grid=grid,
        in_specs=[
            pl.BlockSpec((_BS, 2 * _D), lambda i: (i, 0)),
            pl.BlockSpec((_BS, 2 * _D), lambda i: (i, 0)),
            pl.BlockSpec((_BS, 1), lambda i: (i, 0)),
            pl.BlockSpec((_BS, 1), lambda i: (i, 0)),
            full((2 * _D, 256)), full((1, 256)),
            full((256, 128)), full((1, 128)),
            full((128, _D)), full((1, _D)),
            full((1, _D)), full((1, 1)),
        ],
        out_specs=pl.BlockSpec((_BS,), lambda i: (i,)),
        out_shape=jax.ShapeDtypeStruct((_B,), jnp.float32),
    )(gu, gm, users2d, movies2d,
      W1, b1.reshape(1, 256), W2, b2.reshape(1, 128),
      W3, b3.reshape(1, _D), W4.reshape(1, _D), b4.reshape(1, 1))


def kernel(users, movies, user_table, movie_table,
           W1, b1, W2, b2, W3, b3, W4, b4):
    users = users.astype(jnp.int32)
    movies = movies.astype(jnp.int32)
    ut2 = user_table.reshape(-1, 2 * _D)
    mt2 = movie_table.reshape(-1, 2 * _D)
    gu, gm = _sc_gather(users, movies, ut2, mt2)
    return _tc_mlp(gu, gm, users.reshape(_B, 1), movies.reshape(_B, 1),
                   W1, b1, W2, b2, W3, b3, W4, b4)
